# baseline (speedup 1.0000x reference)
import jax
import jax.numpy as jnp
from jax.experimental import pallas as pl
from jax.experimental.pallas import tpu as pltpu

_BM = 256
_TAU = 1e-4


def _select_col(logits, col):
    num = logits.shape[1]
    onehot = jax.lax.broadcasted_iota(jnp.int32, (1, num), 1) == col
    return jnp.sum(jnp.where(onehot, logits, 0.0), axis=1, keepdims=True)


def _gating_block_kernel(idx_ref, x_ref, w1_ref, b1_ref, w2_ref, b2_ref,
                         wl_ref, bl_ref, out_ref, s1_ref, lg_ref):
    i = pl.program_id(0)
    bm, d = x_ref.shape

    @pl.when(i == 0)
    def _init_s1():
        s1_ref[...] = jnp.sum(w1_ref[...].astype(jnp.float32), axis=0,
                              keepdims=True)

    x = x_ref[...]
    m = jnp.mean(x, axis=1, keepdims=True)
    mb = m.astype(jnp.bfloat16)
    h_c = jax.nn.relu(mb.astype(jnp.float32) * s1_ref[...] + b1_ref[...])
    lg_ref[...] = jnp.dot(h_c.astype(jnp.bfloat16), w2_ref[...],
                          preferred_element_type=jnp.float32) + b2_ref[...]
    logit_c = _select_col(lg_ref[...], idx_ref[0])
    borderline = jnp.any(jnp.abs(logit_c) < _TAU)

    @pl.when(borderline)
    def _exact_gate():
        xp = jnp.broadcast_to(mb, (bm, d))
        h = jax.nn.relu(jnp.dot(xp, w1_ref[...],
                                preferred_element_type=jnp.float32)
                        + b1_ref[...])
        lg_ref[...] = jnp.dot(h.astype(jnp.bfloat16), w2_ref[...],
                              preferred_element_type=jnp.float32) + b2_ref[...]

    gate = jax.nn.sigmoid(_select_col(lg_ref[...], idx_ref[0])) > 0.5

    y = jnp.dot(x.astype(jnp.bfloat16), wl_ref[...],
                preferred_element_type=jnp.float32)
    y = jax.nn.relu(y + bl_ref[...])
    out_ref[...] = jnp.where(gate, y, x)


def kernel(x, W1, b1, W2, b2, Wl, bl, layer_idx):
    n, d = x.shape
    h_dim = W1.shape[1]
    n_layers = W2.shape[1]
    idx = jnp.asarray(layer_idx, jnp.int32).reshape((1,))
    wl_bf = Wl.astype(jnp.bfloat16)
    w1_bf = W1.astype(jnp.bfloat16)
    w2_bf = W2.astype(jnp.bfloat16)
    full = lambda shape: pl.BlockSpec(shape, lambda i, s: tuple(
        0 for _ in shape))

    grid_spec = pltpu.PrefetchScalarGridSpec(
        num_scalar_prefetch=1,
        grid=(n // _BM,),
        in_specs=[
            pl.BlockSpec((_BM, d), lambda i, s: (i, 0)),
            full((d, h_dim)), full((1, h_dim)),
            full((h_dim, n_layers)), full((1, n_layers)),
            full((d, d)), full((1, d)),
        ],
        out_specs=pl.BlockSpec((_BM, d), lambda i, s: (i, 0)),
        scratch_shapes=[pltpu.VMEM((1, h_dim), jnp.float32),
                        pltpu.VMEM((_BM, n_layers), jnp.float32)],
    )
    return pl.pallas_call(
        _gating_block_kernel,
        grid_spec=grid_spec,
        out_shape=jax.ShapeDtypeStruct((n, d), jnp.float32),
    )(idx, x, w1_bf, b1.reshape(1, h_dim), w2_bf, b2.reshape(1, n_layers),
      wl_bf, bl.reshape(1, d))
